# baseline (speedup 1.0000x reference)
import functools

import jax
import jax.numpy as jnp
import numpy as np
from jax.experimental import pallas as pl

K = 10
N = 200
DIM = 512
DEPTH = 4
HEADS = 8
DIM_HEAD = 64
INNER = HEADS * DIM_HEAD
MLP_DIM = 1024
NUM_CLASSES = 40
MAX_VOX = 356
EPS = 1e-5
NTOK = 136

f32 = jnp.float32


def _leaky(x):
    return jnp.where(x > 0, x, 0.2 * x)


VB = 4


def _edge_layer(Xs, Aw, Bw):
    y1s, y2s, pds = [], [], []
    for X in Xs:
        xx = jnp.sum(X * X, axis=1, keepdims=True)
        ones = jnp.ones((N, 1), f32)
        Amat = jnp.concatenate([2.0 * X, -xx, ones], axis=1)
        sign = jnp.where(xx > 0, jnp.float32(1.0), jnp.float32(1e7))
        Bmat = jnp.concatenate([X, ones, -xx], axis=1) * sign
        pd = jax.lax.dot_general(Amat, Bmat, (((1,), (1,)), ((), ())),
                                 preferred_element_type=f32)
        pds.append(pd[None])
        y1s.append(jnp.dot(X, Aw, preferred_element_type=f32))
        y2s.append(jnp.dot(X, Bw, preferred_element_type=f32))

    cur = jnp.concatenate(pds, axis=0)
    iota_f = jax.lax.broadcasted_iota(jnp.int32, (VB, N, N), 2).astype(f32)
    sels = []
    for _ in range(K):
        m = jnp.max(cur, axis=2, keepdims=True)
        t = jnp.where(cur >= m, iota_f, jnp.float32(1e9))
        jmin = jnp.min(t, axis=2, keepdims=True)
        hit = t == jmin
        sels.append(jnp.where(hit, jnp.float32(1.0), jnp.float32(0.0)))
        cur = jnp.where(hit, jnp.float32(-3e38), cur)

    outs = []
    for i in range(VB):
        S = jnp.concatenate([s[i] for s in sels],
                            axis=0).astype(jnp.bfloat16)
        y1 = y1s[i]
        y1h = y1.astype(jnp.bfloat16)
        y1l = (y1 - y1h.astype(f32)).astype(jnp.bfloat16)
        g = (jnp.dot(S, y1h, preferred_element_type=f32)
             + jnp.dot(S, y1l, preferred_element_type=f32))
        mx = jnp.max(g.reshape(K, N, g.shape[1]), axis=0)
        outs.append(_leaky(mx + y2s[i]))
    return outs


def _dgcnn_kernel(x_ref, a1, b1, a2, b2, a3, b3, a4, b4, w5t, s5, b5, out_ref):
    x0 = [x_ref[i] for i in range(VB)]
    x1 = _edge_layer(x0, a1[...], b1[...])
    x2 = _edge_layer(x1, a2[...], b2[...])
    x3 = _edge_layer(x2, a3[...], b3[...])
    x4 = _edge_layer(x3, a4[...], b4[...])
    for i in range(VB):
        cat = jnp.concatenate([x1[i], x2[i], x3[i], x4[i]], axis=1)
        h5 = jnp.dot(cat, w5t[...], preferred_element_type=f32)
        h5 = _leaky(h5 * s5[...] + b5[...])
        pm = jnp.max(h5, axis=0, keepdims=True)
        pa = jnp.mean(h5, axis=0, keepdims=True)
        out_ref[i] = jnp.concatenate([pm, pa], axis=1)


def _voxhead_kernel(p_ref, l1t, s6, b6, l2t, s7, bt7, valid, out_ref):
    h = jnp.dot(p_ref[...], l1t[...], preferred_element_type=f32)
    h = _leaky(h * s6[...] + b6[...])
    h = jnp.dot(h, l2t[...], preferred_element_type=f32)
    h = _leaky(h * s7[...] + bt7[...])
    out_ref[...] = h * valid[...]


def _transformer_kernel(x_in, cnt, ln1g, ln1b, qkvt, aot, aob, ln2g, ln2b,
                        f1t, f1b, f2t, f2b, out_ref):
    l = pl.program_id(0)

    @pl.when(l == 0)
    def _():
        out_ref[...] = x_in[...]

    x = out_ref[...]
    scale = DIM_HEAD ** -0.5

    def ln(v, g, b):
        mu = jnp.mean(v, axis=1, keepdims=True)
        var = jnp.mean((v - mu) ** 2, axis=1, keepdims=True)
        return (v - mu) * jax.lax.rsqrt(var + EPS) * g + b

    h = ln(x, ln1g[0], ln1b[0])
    qkv = jnp.dot(h, qkvt[0], preferred_element_type=f32)

    counts = cnt[...]
    live = counts > 0
    bparts = []
    for b in range(4):
        r0 = b * NTOK
        hparts = []
        for hh in range(HEADS):
            c = hh * DIM_HEAD
            q = qkv[r0:r0 + NTOK, c:c + DIM_HEAD]
            k_ = qkv[r0:r0 + NTOK, INNER + c:INNER + c + DIM_HEAD]
            v = qkv[r0:r0 + NTOK, 2 * INNER + c:2 * INNER + c + DIM_HEAD]
            dots = jax.lax.dot_general(q, k_, (((1,), (1,)), ((), ())),
                                       preferred_element_type=f32) * scale
            dots = jnp.where(live, dots, jnp.float32(-1e30))
            dots = dots - jnp.max(dots, axis=1, keepdims=True)
            e = jnp.exp(dots) * counts
            attn = e / jnp.sum(e, axis=1, keepdims=True)
            hparts.append(jnp.dot(attn, v, preferred_element_type=f32))
        bparts.append(jnp.concatenate(hparts, axis=1))
    o = jnp.concatenate(bparts, axis=0)
    x = x + jnp.dot(o, aot[0], preferred_element_type=f32) + aob[0]

    h = ln(x, ln2g[0], ln2b[0])
    f = jnp.dot(h, f1t[0], preferred_element_type=f32) + f1b[0]
    f = 0.5 * f * (1.0 + jax.lax.erf(f * jnp.float32(0.7071067811865476)))
    x = x + jnp.dot(f, f2t[0], preferred_element_type=f32) + f2b[0]
    out_ref[...] = x


def _head_kernel(x_ref, w1t, cb1, w2t, cb2, out_ref):
    h = _leaky(jnp.dot(x_ref[...], w1t[...], preferred_element_type=f32)
               + cb1[...])
    out_ref[...] = jnp.dot(h, w2t[...], preferred_element_type=f32) + cb2[...]


def _const_spec(shape):
    nd = len(shape)
    return pl.BlockSpec(shape, lambda *_: (0,) * nd)


@jax.jit
def kernel(input, cloud_len_list, voxel_sequence, params):
    p = params
    B, MV = input.shape[0], input.shape[1]
    V = B * MV
    x0 = input.reshape(V, N, 3)

    inv = np.float32(1.0 / np.sqrt(1.0 + EPS))

    def split_w(w):
        c = w.shape[1] // 2
        return w[:, :c].T, (w[:, c:] - w[:, :c]).T

    a1, b1 = split_w(p['conv1_w'])
    a2, b2 = split_w(p['conv2_w'])
    a3, b3 = split_w(p['conv3_w'])
    a4, b4 = split_w(p['conv4_w'])
    w5t = p['conv5_w'].T
    s5 = (p['bn5_g'] * inv)[None]
    b5 = p['bn5_b'][None]

    dg_in = [x0, a1, b1, a2, b2, a3, b3, a4, b4, w5t, s5, b5]
    dg_specs = [pl.BlockSpec((VB, N, 3), lambda v: (v, 0, 0))]
    dg_specs += [_const_spec(t.shape) for t in dg_in[1:]]
    pooled = pl.pallas_call(
        _dgcnn_kernel,
        grid=(V // VB,),
        in_specs=dg_specs,
        out_specs=pl.BlockSpec((VB, 1, 1024), lambda v: (v, 0, 0)),
        out_shape=jax.ShapeDtypeStruct((V, 1, 1024), f32),
    )(*dg_in)
    pooled = pooled.reshape(V, 1024)

    lens = cloud_len_list.astype(jnp.int32)
    valid = (jnp.arange(MV, dtype=jnp.int32)[None, :] < lens[:, None])
    valid = valid.reshape(V, 1).astype(f32)
    vh_in = [pooled, p['lin1_w'].T, (p['bn6_g'] * inv)[None], p['bn6_b'][None],
             p['lin2_w'].T, (p['bn7_g'] * inv)[None],
             (p['lin2_b'] * inv * p['bn7_g'] + p['bn7_b'])[None], valid]
    vf = pl.pallas_call(
        _voxhead_kernel,
        in_specs=[_const_spec(t.shape) for t in vh_in],
        out_specs=_const_spec((V, DIM)),
        out_shape=jax.ShapeDtypeStruct((V, DIM), f32),
    )(*vh_in)

    cls = jnp.broadcast_to(p['cls_token'], (B, 1, DIM))
    pad = jnp.zeros((B, NTOK - 1 - MV, DIM), f32)
    toks = jnp.concatenate([cls, vf.reshape(B, MV, DIM), pad], axis=1)
    xt = toks.reshape(B * NTOK, DIM)

    def d3(t):
        return t[:, None, :] if t.ndim == 2 else t

    counts = jnp.asarray(
        [[1.0] * (1 + MV) + [float(MAX_VOX - MV)] + [0.0] * (NTOK - MV - 2)],
        dtype=f32)
    tr_in = [xt, counts,
             d3(p['ln1_g']), d3(p['ln1_b']),
             jnp.transpose(p['qkv_w'], (0, 2, 1)),
             jnp.transpose(p['attn_out_w'], (0, 2, 1)), d3(p['attn_out_b']),
             d3(p['ln2_g']), d3(p['ln2_b']),
             jnp.transpose(p['ff_w1'], (0, 2, 1)), d3(p['ff_b1']),
             jnp.transpose(p['ff_w2'], (0, 2, 1)), d3(p['ff_b2'])]
    tr_specs = [_const_spec((B * NTOK, DIM)), _const_spec((1, NTOK))]
    for t in tr_in[2:]:
        tr_specs.append(
            pl.BlockSpec((1,) + t.shape[1:], lambda l: (l, 0, 0)))
    xo = pl.pallas_call(
        _transformer_kernel,
        grid=(DEPTH,),
        in_specs=tr_specs,
        out_specs=_const_spec((B * NTOK, DIM)),
        out_shape=jax.ShapeDtypeStruct((B * NTOK, DIM), f32),
    )(*tr_in)

    cls_f = xo.reshape(B, NTOK, DIM)[:, 0]
    cls_p = jnp.zeros((8, DIM), f32).at[:B].set(cls_f)
    hd_in = [cls_p, p['cls_w1'].T, p['cls_b1'][None],
             p['cls_w2'].T, p['cls_b2'][None]]
    logits = pl.pallas_call(
        _head_kernel,
        in_specs=[_const_spec(t.shape) for t in hd_in],
        out_specs=_const_spec((8, NUM_CLASSES)),
        out_shape=jax.ShapeDtypeStruct((8, NUM_CLASSES), f32),
    )(*hd_in)
    return logits[:B]


# speedup vs baseline: 1.1207x; 1.1207x over previous
import functools

import jax
import jax.numpy as jnp
import numpy as np
from jax.experimental import pallas as pl

K = 10
N = 200
DIM = 512
DEPTH = 4
HEADS = 8
DIM_HEAD = 64
INNER = HEADS * DIM_HEAD
MLP_DIM = 1024
NUM_CLASSES = 40
MAX_VOX = 356
EPS = 1e-5
NTOK = 136

f32 = jnp.float32


def _leaky(x):
    return jnp.where(x > 0, x, 0.2 * x)


VB = 2


def _edge_layer(Xs, Aw, Bw):
    y1s, y2s, pds = [], [], []
    for X in Xs:
        xx = jnp.sum(X * X, axis=1, keepdims=True)
        ones = jnp.ones((N, 1), f32)
        Amat = jnp.concatenate([2.0 * X, -xx, ones], axis=1)
        sign = jnp.where(xx > 0, jnp.float32(1.0), jnp.float32(1e7))
        Bmat = jnp.concatenate([X, ones, -xx], axis=1) * sign
        pd = jax.lax.dot_general(Amat, Bmat, (((1,), (1,)), ((), ())),
                                 preferred_element_type=f32)
        pds.append(pd[None])
        y1s.append(jnp.dot(X, Aw, preferred_element_type=f32))
        y2s.append(jnp.dot(X, Bw, preferred_element_type=f32))

    cur = jnp.concatenate(pds, axis=0)
    iota_f = jax.lax.broadcasted_iota(jnp.int32, (VB, N, N), 2).astype(f32)
    sels = []
    for _ in range(K):
        m = jnp.max(cur, axis=2, keepdims=True)
        t = jnp.where(cur >= m, iota_f, jnp.float32(1e9))
        jmin = jnp.min(t, axis=2, keepdims=True)
        hit = t == jmin
        sels.append(jnp.where(hit, jnp.float32(1.0), jnp.float32(0.0)))
        cur = jnp.where(hit, jnp.float32(-3e38), cur)

    outs = []
    for i in range(VB):
        S = jnp.concatenate([s[i] for s in sels], axis=0)
        g = jnp.dot(S, y1s[i], preferred_element_type=f32)
        mx = jnp.max(g.reshape(K, N, g.shape[1]), axis=0)
        outs.append(_leaky(mx + y2s[i]))
    return outs


def _dgcnn_kernel(x_ref, a1, b1, a2, b2, a3, b3, a4, b4, w5t, s5, b5, out_ref):
    x0 = [x_ref[i] for i in range(VB)]
    x1 = _edge_layer(x0, a1[...], b1[...])
    x2 = _edge_layer(x1, a2[...], b2[...])
    x3 = _edge_layer(x2, a3[...], b3[...])
    x4 = _edge_layer(x3, a4[...], b4[...])
    for i in range(VB):
        cat = jnp.concatenate([x1[i], x2[i], x3[i], x4[i]], axis=1)
        h5 = jnp.dot(cat, w5t[...], preferred_element_type=f32)
        h5 = _leaky(h5 * s5[...] + b5[...])
        pm = jnp.max(h5, axis=0, keepdims=True)
        pa = jnp.mean(h5, axis=0, keepdims=True)
        out_ref[i] = jnp.concatenate([pm, pa], axis=1)


def _voxhead_kernel(p_ref, l1t, s6, b6, l2t, s7, bt7, valid, out_ref):
    h = jnp.dot(p_ref[...], l1t[...], preferred_element_type=f32)
    h = _leaky(h * s6[...] + b6[...])
    h = jnp.dot(h, l2t[...], preferred_element_type=f32)
    h = _leaky(h * s7[...] + bt7[...])
    out_ref[...] = h * valid[...]


def _transformer_kernel(x_in, cnt, ln1g, ln1b, qkvt, aot, aob, ln2g, ln2b,
                        f1t, f1b, f2t, f2b, out_ref):
    l = pl.program_id(0)

    @pl.when(l == 0)
    def _():
        out_ref[...] = x_in[...]

    x = out_ref[...]
    scale = DIM_HEAD ** -0.5

    def ln(v, g, b):
        mu = jnp.mean(v, axis=1, keepdims=True)
        var = jnp.mean((v - mu) ** 2, axis=1, keepdims=True)
        return (v - mu) * jax.lax.rsqrt(var + EPS) * g + b

    h = ln(x, ln1g[0], ln1b[0])
    qkv = jnp.dot(h, qkvt[0], preferred_element_type=f32)

    counts = cnt[...]
    live = counts > 0
    bparts = []
    for b in range(4):
        r0 = b * NTOK
        hparts = []
        for hh in range(HEADS):
            c = hh * DIM_HEAD
            q = qkv[r0:r0 + NTOK, c:c + DIM_HEAD]
            k_ = qkv[r0:r0 + NTOK, INNER + c:INNER + c + DIM_HEAD]
            v = qkv[r0:r0 + NTOK, 2 * INNER + c:2 * INNER + c + DIM_HEAD]
            dots = jax.lax.dot_general(q, k_, (((1,), (1,)), ((), ())),
                                       preferred_element_type=f32) * scale
            dots = jnp.where(live, dots, jnp.float32(-1e30))
            dots = dots - jnp.max(dots, axis=1, keepdims=True)
            e = jnp.exp(dots) * counts
            attn = e / jnp.sum(e, axis=1, keepdims=True)
            hparts.append(jnp.dot(attn, v, preferred_element_type=f32))
        bparts.append(jnp.concatenate(hparts, axis=1))
    o = jnp.concatenate(bparts, axis=0)
    x = x + jnp.dot(o, aot[0], preferred_element_type=f32) + aob[0]

    h = ln(x, ln2g[0], ln2b[0])
    f = jnp.dot(h, f1t[0], preferred_element_type=f32) + f1b[0]
    f = 0.5 * f * (1.0 + jax.lax.erf(f * jnp.float32(0.7071067811865476)))
    x = x + jnp.dot(f, f2t[0], preferred_element_type=f32) + f2b[0]
    out_ref[...] = x


def _head_kernel(x_ref, w1t, cb1, w2t, cb2, out_ref):
    h = _leaky(jnp.dot(x_ref[...], w1t[...], preferred_element_type=f32)
               + cb1[...])
    out_ref[...] = jnp.dot(h, w2t[...], preferred_element_type=f32) + cb2[...]


def _const_spec(shape):
    nd = len(shape)
    return pl.BlockSpec(shape, lambda *_: (0,) * nd)


@jax.jit
def kernel(input, cloud_len_list, voxel_sequence, params):
    p = params
    B, MV = input.shape[0], input.shape[1]
    V = B * MV
    x0 = input.reshape(V, N, 3)

    inv = np.float32(1.0 / np.sqrt(1.0 + EPS))

    def split_w(w):
        c = w.shape[1] // 2
        return w[:, :c].T, (w[:, c:] - w[:, :c]).T

    a1, b1 = split_w(p['conv1_w'])
    a2, b2 = split_w(p['conv2_w'])
    a3, b3 = split_w(p['conv3_w'])
    a4, b4 = split_w(p['conv4_w'])
    w5t = p['conv5_w'].T
    s5 = (p['bn5_g'] * inv)[None]
    b5 = p['bn5_b'][None]

    dg_in = [x0, a1, b1, a2, b2, a3, b3, a4, b4, w5t, s5, b5]
    dg_specs = [pl.BlockSpec((VB, N, 3), lambda v: (v, 0, 0))]
    dg_specs += [_const_spec(t.shape) for t in dg_in[1:]]
    pooled = pl.pallas_call(
        _dgcnn_kernel,
        grid=(V // VB,),
        in_specs=dg_specs,
        out_specs=pl.BlockSpec((VB, 1, 1024), lambda v: (v, 0, 0)),
        out_shape=jax.ShapeDtypeStruct((V, 1, 1024), f32),
    )(*dg_in)
    pooled = pooled.reshape(V, 1024)

    lens = cloud_len_list.astype(jnp.int32)
    valid = (jnp.arange(MV, dtype=jnp.int32)[None, :] < lens[:, None])
    valid = valid.reshape(V, 1).astype(f32)
    vh_in = [pooled, p['lin1_w'].T, (p['bn6_g'] * inv)[None], p['bn6_b'][None],
             p['lin2_w'].T, (p['bn7_g'] * inv)[None],
             (p['lin2_b'] * inv * p['bn7_g'] + p['bn7_b'])[None], valid]
    vf = pl.pallas_call(
        _voxhead_kernel,
        in_specs=[_const_spec(t.shape) for t in vh_in],
        out_specs=_const_spec((V, DIM)),
        out_shape=jax.ShapeDtypeStruct((V, DIM), f32),
    )(*vh_in)

    cls = jnp.broadcast_to(p['cls_token'], (B, 1, DIM))
    pad = jnp.zeros((B, NTOK - 1 - MV, DIM), f32)
    toks = jnp.concatenate([cls, vf.reshape(B, MV, DIM), pad], axis=1)
    xt = toks.reshape(B * NTOK, DIM)

    def d3(t):
        return t[:, None, :] if t.ndim == 2 else t

    counts = jnp.asarray(
        [[1.0] * (1 + MV) + [float(MAX_VOX - MV)] + [0.0] * (NTOK - MV - 2)],
        dtype=f32)
    tr_in = [xt, counts,
             d3(p['ln1_g']), d3(p['ln1_b']),
             jnp.transpose(p['qkv_w'], (0, 2, 1)),
             jnp.transpose(p['attn_out_w'], (0, 2, 1)), d3(p['attn_out_b']),
             d3(p['ln2_g']), d3(p['ln2_b']),
             jnp.transpose(p['ff_w1'], (0, 2, 1)), d3(p['ff_b1']),
             jnp.transpose(p['ff_w2'], (0, 2, 1)), d3(p['ff_b2'])]
    tr_specs = [_const_spec((B * NTOK, DIM)), _const_spec((1, NTOK))]
    for t in tr_in[2:]:
        tr_specs.append(
            pl.BlockSpec((1,) + t.shape[1:], lambda l: (l, 0, 0)))
    xo = pl.pallas_call(
        _transformer_kernel,
        grid=(DEPTH,),
        in_specs=tr_specs,
        out_specs=_const_spec((B * NTOK, DIM)),
        out_shape=jax.ShapeDtypeStruct((B * NTOK, DIM), f32),
    )(*tr_in)

    cls_f = xo.reshape(B, NTOK, DIM)[:, 0]
    cls_p = jnp.zeros((8, DIM), f32).at[:B].set(cls_f)
    hd_in = [cls_p, p['cls_w1'].T, p['cls_b1'][None],
             p['cls_w2'].T, p['cls_b2'][None]]
    logits = pl.pallas_call(
        _head_kernel,
        in_specs=[_const_spec(t.shape) for t in hd_in],
        out_specs=_const_spec((8, NUM_CLASSES)),
        out_shape=jax.ShapeDtypeStruct((8, NUM_CLASSES), f32),
    )(*hd_in)
    return logits[:B]


# speedup vs baseline: 1.5045x; 1.3425x over previous
import functools

import jax
import jax.numpy as jnp
import numpy as np
from jax.experimental import pallas as pl

K = 10
N = 200
DIM = 512
DEPTH = 4
HEADS = 8
DIM_HEAD = 64
INNER = HEADS * DIM_HEAD
MLP_DIM = 1024
NUM_CLASSES = 40
MAX_VOX = 356
EPS = 1e-5
NTOK = 136

f32 = jnp.float32


def _leaky(x):
    return jnp.where(x > 0, x, 0.2 * x)


VB = 4


def _edge_layer(Xs, Aw, Bw):
    y1s, y2s, pds = [], [], []
    for X in Xs:
        xx = jnp.sum(X * X, axis=1, keepdims=True)
        ones = jnp.ones((N, 1), f32)
        Amat = jnp.concatenate([2.0 * X, -xx, ones], axis=1)
        sign = jnp.where(xx > 0, jnp.float32(1.0), jnp.float32(1e7))
        Bmat = jnp.concatenate([X, ones, -xx], axis=1) * sign
        pd = jax.lax.dot_general(Amat, Bmat, (((1,), (1,)), ((), ())),
                                 preferred_element_type=f32)
        pds.append(pd[None])
        y1s.append(jnp.dot(X, Aw, preferred_element_type=f32))
        y2s.append(jnp.dot(X, Bw, preferred_element_type=f32))

    cur = jnp.concatenate(pds, axis=0)
    iota_f = jax.lax.broadcasted_iota(jnp.int32, (VB, N, N), 2).astype(f32)
    diag = (jax.lax.broadcasted_iota(jnp.int32, (VB, N, N), 2)
            == jax.lax.broadcasted_iota(jnp.int32, (VB, N, N), 1))
    cur = jnp.where(diag, jnp.float32(-3e38), cur)
    sels = []
    for _ in range(K - 1):
        m = jnp.max(cur, axis=2, keepdims=True)
        t = jnp.where(cur >= m, iota_f, jnp.float32(1e9))
        jmin = jnp.min(t, axis=2, keepdims=True)
        hit = t == jmin
        sels.append(jnp.where(hit, jnp.float32(1.0), jnp.float32(0.0)))
        cur = jnp.where(hit, jnp.float32(-3e38), cur)

    outs = []
    for i in range(VB):
        S = jnp.concatenate([s[i] for s in sels], axis=0)
        g = jnp.dot(S, y1s[i], preferred_element_type=f32)
        mx = jnp.max(g.reshape(K - 1, N, g.shape[1]), axis=0)
        outs.append(_leaky(jnp.maximum(mx, y1s[i]) + y2s[i]))
    return outs


def _dgcnn_kernel(x_ref, a1, b1, a2, b2, a3, b3, a4, b4, w5t, s5, b5, out_ref):
    x0 = [x_ref[i] for i in range(VB)]
    x1 = _edge_layer(x0, a1[...], b1[...])
    x2 = _edge_layer(x1, a2[...], b2[...])
    x3 = _edge_layer(x2, a3[...], b3[...])
    x4 = _edge_layer(x3, a4[...], b4[...])
    for i in range(VB):
        cat = jnp.concatenate([x1[i], x2[i], x3[i], x4[i]], axis=1)
        h5 = jnp.dot(cat, w5t[...], preferred_element_type=f32)
        h5 = _leaky(h5 * s5[...] + b5[...])
        pm = jnp.max(h5, axis=0, keepdims=True)
        pa = jnp.mean(h5, axis=0, keepdims=True)
        out_ref[i] = jnp.concatenate([pm, pa], axis=1)


def _voxhead_kernel(p_ref, l1t, s6, b6, l2t, s7, bt7, valid, out_ref):
    h = jnp.dot(p_ref[...], l1t[...], preferred_element_type=f32)
    h = _leaky(h * s6[...] + b6[...])
    h = jnp.dot(h, l2t[...], preferred_element_type=f32)
    h = _leaky(h * s7[...] + bt7[...])
    out_ref[...] = h * valid[...]


def _transformer_kernel(x_in, cnt, ln1g, ln1b, qkvt, aot, aob, ln2g, ln2b,
                        f1t, f1b, f2t, f2b, out_ref):
    l = pl.program_id(0)

    @pl.when(l == 0)
    def _():
        out_ref[...] = x_in[...]

    x = out_ref[...]
    scale = DIM_HEAD ** -0.5

    def ln(v, g, b):
        mu = jnp.mean(v, axis=1, keepdims=True)
        var = jnp.mean((v - mu) ** 2, axis=1, keepdims=True)
        return (v - mu) * jax.lax.rsqrt(var + EPS) * g + b

    h = ln(x, ln1g[0], ln1b[0])
    qkv = jnp.dot(h, qkvt[0], preferred_element_type=f32)

    counts = cnt[...]
    live = counts > 0
    bparts = []
    for b in range(4):
        r0 = b * NTOK
        hparts = []
        for hh in range(HEADS):
            c = hh * DIM_HEAD
            q = qkv[r0:r0 + NTOK, c:c + DIM_HEAD]
            k_ = qkv[r0:r0 + NTOK, INNER + c:INNER + c + DIM_HEAD]
            v = qkv[r0:r0 + NTOK, 2 * INNER + c:2 * INNER + c + DIM_HEAD]
            dots = jax.lax.dot_general(q, k_, (((1,), (1,)), ((), ())),
                                       preferred_element_type=f32) * scale
            dots = jnp.where(live, dots, jnp.float32(-1e30))
            dots = dots - jnp.max(dots, axis=1, keepdims=True)
            e = jnp.exp(dots) * counts
            attn = e / jnp.sum(e, axis=1, keepdims=True)
            hparts.append(jnp.dot(attn, v, preferred_element_type=f32))
        bparts.append(jnp.concatenate(hparts, axis=1))
    o = jnp.concatenate(bparts, axis=0)
    x = x + jnp.dot(o, aot[0], preferred_element_type=f32) + aob[0]

    h = ln(x, ln2g[0], ln2b[0])
    f = jnp.dot(h, f1t[0], preferred_element_type=f32) + f1b[0]
    f = 0.5 * f * (1.0 + jax.lax.erf(f * jnp.float32(0.7071067811865476)))
    x = x + jnp.dot(f, f2t[0], preferred_element_type=f32) + f2b[0]
    out_ref[...] = x


def _head_kernel(x_ref, w1t, cb1, w2t, cb2, out_ref):
    h = _leaky(jnp.dot(x_ref[...], w1t[...], preferred_element_type=f32)
               + cb1[...])
    out_ref[...] = jnp.dot(h, w2t[...], preferred_element_type=f32) + cb2[...]


def _const_spec(shape):
    nd = len(shape)
    return pl.BlockSpec(shape, lambda *_: (0,) * nd)


@jax.jit
def kernel(input, cloud_len_list, voxel_sequence, params):
    p = params
    B, MV = input.shape[0], input.shape[1]
    V = B * MV
    x0 = input.reshape(V, N, 3)

    inv = np.float32(1.0 / np.sqrt(1.0 + EPS))

    def split_w(w):
        c = w.shape[1] // 2
        return w[:, :c].T, (w[:, c:] - w[:, :c]).T

    a1, b1 = split_w(p['conv1_w'])
    a2, b2 = split_w(p['conv2_w'])
    a3, b3 = split_w(p['conv3_w'])
    a4, b4 = split_w(p['conv4_w'])
    w5t = p['conv5_w'].T
    s5 = (p['bn5_g'] * inv)[None]
    b5 = p['bn5_b'][None]

    dg_in = [x0, a1, b1, a2, b2, a3, b3, a4, b4, w5t, s5, b5]
    dg_specs = [pl.BlockSpec((VB, N, 3), lambda v: (v, 0, 0))]
    dg_specs += [_const_spec(t.shape) for t in dg_in[1:]]
    pooled = pl.pallas_call(
        _dgcnn_kernel,
        grid=(V // VB,),
        in_specs=dg_specs,
        out_specs=pl.BlockSpec((VB, 1, 1024), lambda v: (v, 0, 0)),
        out_shape=jax.ShapeDtypeStruct((V, 1, 1024), f32),
    )(*dg_in)
    pooled = pooled.reshape(V, 1024)

    lens = cloud_len_list.astype(jnp.int32)
    valid = (jnp.arange(MV, dtype=jnp.int32)[None, :] < lens[:, None])
    valid = valid.reshape(V, 1).astype(f32)
    vh_in = [pooled, p['lin1_w'].T, (p['bn6_g'] * inv)[None], p['bn6_b'][None],
             p['lin2_w'].T, (p['bn7_g'] * inv)[None],
             (p['lin2_b'] * inv * p['bn7_g'] + p['bn7_b'])[None], valid]
    vf = pl.pallas_call(
        _voxhead_kernel,
        in_specs=[_const_spec(t.shape) for t in vh_in],
        out_specs=_const_spec((V, DIM)),
        out_shape=jax.ShapeDtypeStruct((V, DIM), f32),
    )(*vh_in)

    cls = jnp.broadcast_to(p['cls_token'], (B, 1, DIM))
    pad = jnp.zeros((B, NTOK - 1 - MV, DIM), f32)
    toks = jnp.concatenate([cls, vf.reshape(B, MV, DIM), pad], axis=1)
    xt = toks.reshape(B * NTOK, DIM)

    def d3(t):
        return t[:, None, :] if t.ndim == 2 else t

    counts = jnp.asarray(
        [[1.0] * (1 + MV) + [float(MAX_VOX - MV)] + [0.0] * (NTOK - MV - 2)],
        dtype=f32)
    tr_in = [xt, counts,
             d3(p['ln1_g']), d3(p['ln1_b']),
             jnp.transpose(p['qkv_w'], (0, 2, 1)),
             jnp.transpose(p['attn_out_w'], (0, 2, 1)), d3(p['attn_out_b']),
             d3(p['ln2_g']), d3(p['ln2_b']),
             jnp.transpose(p['ff_w1'], (0, 2, 1)), d3(p['ff_b1']),
             jnp.transpose(p['ff_w2'], (0, 2, 1)), d3(p['ff_b2'])]
    tr_specs = [_const_spec((B * NTOK, DIM)), _const_spec((1, NTOK))]
    for t in tr_in[2:]:
        tr_specs.append(
            pl.BlockSpec((1,) + t.shape[1:], lambda l: (l, 0, 0)))
    xo = pl.pallas_call(
        _transformer_kernel,
        grid=(DEPTH,),
        in_specs=tr_specs,
        out_specs=_const_spec((B * NTOK, DIM)),
        out_shape=jax.ShapeDtypeStruct((B * NTOK, DIM), f32),
    )(*tr_in)

    cls_f = xo.reshape(B, NTOK, DIM)[:, 0]
    cls_p = jnp.zeros((8, DIM), f32).at[:B].set(cls_f)
    hd_in = [cls_p, p['cls_w1'].T, p['cls_b1'][None],
             p['cls_w2'].T, p['cls_b2'][None]]
    logits = pl.pallas_call(
        _head_kernel,
        in_specs=[_const_spec(t.shape) for t in hd_in],
        out_specs=_const_spec((8, NUM_CLASSES)),
        out_shape=jax.ShapeDtypeStruct((8, NUM_CLASSES), f32),
    )(*hd_in)
    return logits[:B]


# speedup vs baseline: 1.5082x; 1.0025x over previous
import functools

import jax
import jax.numpy as jnp
import numpy as np
from jax.experimental import pallas as pl
from jax.experimental.pallas import tpu as pltpu

K = 10
N = 200
DIM = 512
DEPTH = 4
HEADS = 8
DIM_HEAD = 64
INNER = HEADS * DIM_HEAD
MLP_DIM = 1024
NUM_CLASSES = 40
MAX_VOX = 356
EPS = 1e-5
NTOK = 136

f32 = jnp.float32


def _leaky(x):
    return jnp.where(x > 0, x, 0.2 * x)


VB = 4


def _edge_layer(Xs, Aw, Bw):
    y1s, y2s, pds = [], [], []
    for X in Xs:
        xx = jnp.sum(X * X, axis=1, keepdims=True)
        ones = jnp.ones((N, 1), f32)
        Amat = jnp.concatenate([2.0 * X, -xx, ones], axis=1)
        sign = jnp.where(xx > 0, jnp.float32(1.0), jnp.float32(1e7))
        Bmat = jnp.concatenate([X, ones, -xx], axis=1) * sign
        pd = jax.lax.dot_general(Amat, Bmat, (((1,), (1,)), ((), ())),
                                 preferred_element_type=f32)
        pds.append(pd[None])
        y1s.append(jnp.dot(X, Aw, preferred_element_type=f32))
        y2s.append(jnp.dot(X, Bw, preferred_element_type=f32))

    cur = jnp.concatenate(pds, axis=0)
    iota_f = jax.lax.broadcasted_iota(jnp.int32, (VB, N, N), 2).astype(f32)
    diag = (jax.lax.broadcasted_iota(jnp.int32, (VB, N, N), 2)
            == jax.lax.broadcasted_iota(jnp.int32, (VB, N, N), 1))
    cur = jnp.where(diag, jnp.float32(-3e38), cur)
    sels = []
    for _ in range(K - 1):
        m = jnp.max(cur, axis=2, keepdims=True)
        t = jnp.where(cur >= m, iota_f, jnp.float32(1e9))
        jmin = jnp.min(t, axis=2, keepdims=True)
        hit = t == jmin
        sels.append(jnp.where(hit, jnp.float32(1.0), jnp.float32(0.0)))
        cur = jnp.where(hit, jnp.float32(-3e38), cur)

    outs = []
    for i in range(VB):
        S = jnp.concatenate([s[i] for s in sels], axis=0)
        g = jnp.dot(S, y1s[i], preferred_element_type=f32)
        mx = jnp.max(g.reshape(K - 1, N, g.shape[1]), axis=0)
        outs.append(_leaky(jnp.maximum(mx, y1s[i]) + y2s[i]))
    return outs


def _dgcnn_kernel(x_ref, a1, b1, a2, b2, a3, b3, a4, b4, w5t, s5, b5, out_ref):
    x0 = [x_ref[i] for i in range(VB)]
    x1 = _edge_layer(x0, a1[...], b1[...])
    x2 = _edge_layer(x1, a2[...], b2[...])
    x3 = _edge_layer(x2, a3[...], b3[...])
    x4 = _edge_layer(x3, a4[...], b4[...])
    for i in range(VB):
        cat = jnp.concatenate([x1[i], x2[i], x3[i], x4[i]], axis=1)
        h5 = jnp.dot(cat, w5t[...], preferred_element_type=f32)
        h5 = _leaky(h5 * s5[...] + b5[...])
        pm = jnp.max(h5, axis=0, keepdims=True)
        pa = jnp.mean(h5, axis=0, keepdims=True)
        out_ref[i] = jnp.concatenate([pm, pa], axis=1)


def _voxhead_kernel(p_ref, l1t, s6, b6, l2t, s7, bt7, valid, out_ref):
    h = jnp.dot(p_ref[...], l1t[...], preferred_element_type=f32)
    h = _leaky(h * s6[...] + b6[...])
    h = jnp.dot(h, l2t[...], preferred_element_type=f32)
    h = _leaky(h * s7[...] + bt7[...])
    out_ref[...] = h * valid[...]


def _transformer_kernel(x_in, cnt, ln1g, ln1b, qkvt, aot, aob, ln2g, ln2b,
                        f1t, f1b, f2t, f2b, out_ref):
    l = pl.program_id(0)

    @pl.when(l == 0)
    def _():
        out_ref[...] = x_in[...]

    x = out_ref[...]
    scale = DIM_HEAD ** -0.5

    def ln(v, g, b):
        mu = jnp.mean(v, axis=1, keepdims=True)
        var = jnp.mean((v - mu) ** 2, axis=1, keepdims=True)
        return (v - mu) * jax.lax.rsqrt(var + EPS) * g + b

    h = ln(x, ln1g[0], ln1b[0])
    qkv = jnp.dot(h, qkvt[0], preferred_element_type=f32)

    counts = cnt[...]
    live = counts > 0
    bparts = []
    for b in range(4):
        r0 = b * NTOK
        hparts = []
        for hh in range(HEADS):
            c = hh * DIM_HEAD
            q = qkv[r0:r0 + NTOK, c:c + DIM_HEAD]
            k_ = qkv[r0:r0 + NTOK, INNER + c:INNER + c + DIM_HEAD]
            v = qkv[r0:r0 + NTOK, 2 * INNER + c:2 * INNER + c + DIM_HEAD]
            dots = jax.lax.dot_general(q, k_, (((1,), (1,)), ((), ())),
                                       preferred_element_type=f32) * scale
            dots = jnp.where(live, dots, jnp.float32(-1e30))
            dots = dots - jnp.max(dots, axis=1, keepdims=True)
            e = jnp.exp(dots) * counts
            attn = e / jnp.sum(e, axis=1, keepdims=True)
            hparts.append(jnp.dot(attn, v, preferred_element_type=f32))
        bparts.append(jnp.concatenate(hparts, axis=1))
    o = jnp.concatenate(bparts, axis=0)
    x = x + jnp.dot(o, aot[0], preferred_element_type=f32) + aob[0]

    h = ln(x, ln2g[0], ln2b[0])
    f = jnp.dot(h, f1t[0], preferred_element_type=f32) + f1b[0]
    f = 0.5 * f * (1.0 + jax.lax.erf(f * jnp.float32(0.7071067811865476)))
    x = x + jnp.dot(f, f2t[0], preferred_element_type=f32) + f2b[0]
    out_ref[...] = x


def _head_kernel(x_ref, w1t, cb1, w2t, cb2, out_ref):
    h = _leaky(jnp.dot(x_ref[...], w1t[...], preferred_element_type=f32)
               + cb1[...])
    out_ref[...] = jnp.dot(h, w2t[...], preferred_element_type=f32) + cb2[...]


def _const_spec(shape):
    nd = len(shape)
    return pl.BlockSpec(shape, lambda *_: (0,) * nd)


@jax.jit
def kernel(input, cloud_len_list, voxel_sequence, params):
    p = params
    B, MV = input.shape[0], input.shape[1]
    V = B * MV
    x0 = input.reshape(V, N, 3)

    inv = np.float32(1.0 / np.sqrt(1.0 + EPS))

    def split_w(w):
        c = w.shape[1] // 2
        return w[:, :c].T, (w[:, c:] - w[:, :c]).T

    a1, b1 = split_w(p['conv1_w'])
    a2, b2 = split_w(p['conv2_w'])
    a3, b3 = split_w(p['conv3_w'])
    a4, b4 = split_w(p['conv4_w'])
    w5t = p['conv5_w'].T
    s5 = (p['bn5_g'] * inv)[None]
    b5 = p['bn5_b'][None]

    dg_in = [x0, a1, b1, a2, b2, a3, b3, a4, b4, w5t, s5, b5]
    dg_specs = [pl.BlockSpec((VB, N, 3), lambda v: (v, 0, 0))]
    dg_specs += [_const_spec(t.shape) for t in dg_in[1:]]
    pooled = pl.pallas_call(
        _dgcnn_kernel,
        grid=(V // VB,),
        compiler_params=pltpu.CompilerParams(
            dimension_semantics=("parallel",)),
        in_specs=dg_specs,
        out_specs=pl.BlockSpec((VB, 1, 1024), lambda v: (v, 0, 0)),
        out_shape=jax.ShapeDtypeStruct((V, 1, 1024), f32),
    )(*dg_in)
    pooled = pooled.reshape(V, 1024)

    lens = cloud_len_list.astype(jnp.int32)
    valid = (jnp.arange(MV, dtype=jnp.int32)[None, :] < lens[:, None])
    valid = valid.reshape(V, 1).astype(f32)
    vh_in = [pooled, p['lin1_w'].T, (p['bn6_g'] * inv)[None], p['bn6_b'][None],
             p['lin2_w'].T, (p['bn7_g'] * inv)[None],
             (p['lin2_b'] * inv * p['bn7_g'] + p['bn7_b'])[None], valid]
    vf = pl.pallas_call(
        _voxhead_kernel,
        in_specs=[_const_spec(t.shape) for t in vh_in],
        out_specs=_const_spec((V, DIM)),
        out_shape=jax.ShapeDtypeStruct((V, DIM), f32),
    )(*vh_in)

    cls = jnp.broadcast_to(p['cls_token'], (B, 1, DIM))
    pad = jnp.zeros((B, NTOK - 1 - MV, DIM), f32)
    toks = jnp.concatenate([cls, vf.reshape(B, MV, DIM), pad], axis=1)
    xt = toks.reshape(B * NTOK, DIM)

    def d3(t):
        return t[:, None, :] if t.ndim == 2 else t

    counts = jnp.asarray(
        [[1.0] * (1 + MV) + [float(MAX_VOX - MV)] + [0.0] * (NTOK - MV - 2)],
        dtype=f32)
    tr_in = [xt, counts,
             d3(p['ln1_g']), d3(p['ln1_b']),
             jnp.transpose(p['qkv_w'], (0, 2, 1)),
             jnp.transpose(p['attn_out_w'], (0, 2, 1)), d3(p['attn_out_b']),
             d3(p['ln2_g']), d3(p['ln2_b']),
             jnp.transpose(p['ff_w1'], (0, 2, 1)), d3(p['ff_b1']),
             jnp.transpose(p['ff_w2'], (0, 2, 1)), d3(p['ff_b2'])]
    tr_specs = [_const_spec((B * NTOK, DIM)), _const_spec((1, NTOK))]
    for t in tr_in[2:]:
        tr_specs.append(
            pl.BlockSpec((1,) + t.shape[1:], lambda l: (l, 0, 0)))
    xo = pl.pallas_call(
        _transformer_kernel,
        grid=(DEPTH,),
        in_specs=tr_specs,
        out_specs=_const_spec((B * NTOK, DIM)),
        out_shape=jax.ShapeDtypeStruct((B * NTOK, DIM), f32),
    )(*tr_in)

    cls_f = xo.reshape(B, NTOK, DIM)[:, 0]
    cls_p = jnp.zeros((8, DIM), f32).at[:B].set(cls_f)
    hd_in = [cls_p, p['cls_w1'].T, p['cls_b1'][None],
             p['cls_w2'].T, p['cls_b2'][None]]
    logits = pl.pallas_call(
        _head_kernel,
        in_specs=[_const_spec(t.shape) for t in hd_in],
        out_specs=_const_spec((8, NUM_CLASSES)),
        out_shape=jax.ShapeDtypeStruct((8, NUM_CLASSES), f32),
    )(*hd_in)
    return logits[:B]


# speedup vs baseline: 1.5093x; 1.0007x over previous
import functools

import jax
import jax.numpy as jnp
import numpy as np
from jax.experimental import pallas as pl
from jax.experimental.pallas import tpu as pltpu

K = 10
N = 200
DIM = 512
DEPTH = 4
HEADS = 8
DIM_HEAD = 64
INNER = HEADS * DIM_HEAD
MLP_DIM = 1024
NUM_CLASSES = 40
MAX_VOX = 356
EPS = 1e-5
NTOK = 136

f32 = jnp.float32


def _leaky(x):
    return jnp.where(x > 0, x, 0.2 * x)


VB = 4


def _edge_layer(Xs, Aw, Bw):
    y1s, y2s, pds = [], [], []
    for X in Xs:
        xx = jnp.sum(X * X, axis=1, keepdims=True)
        ones = jnp.ones((N, 1), f32)
        Amat = jnp.concatenate([2.0 * X, -xx, ones], axis=1)
        sign = jnp.where(xx > 0, jnp.float32(1.0), jnp.float32(1e7))
        Bmat = jnp.concatenate([X, ones, -xx], axis=1) * sign
        pd = jax.lax.dot_general(Amat, Bmat, (((1,), (1,)), ((), ())),
                                 preferred_element_type=f32)
        pds.append(pd[None])
        y1s.append(jax.lax.dot_general(X, Aw, (((1,), (1,)), ((), ())),
                                       preferred_element_type=f32))
        y2s.append(jax.lax.dot_general(X, Bw, (((1,), (1,)), ((), ())),
                                       preferred_element_type=f32))

    cur = jnp.concatenate(pds, axis=0)
    iota_f = jax.lax.broadcasted_iota(jnp.int32, (VB, N, N), 2).astype(f32)
    diag = (jax.lax.broadcasted_iota(jnp.int32, (VB, N, N), 2)
            == jax.lax.broadcasted_iota(jnp.int32, (VB, N, N), 1))
    cur = jnp.where(diag, jnp.float32(-3e38), cur)
    sels = []
    for _ in range(K - 1):
        m = jnp.max(cur, axis=2, keepdims=True)
        t = jnp.where(cur >= m, iota_f, jnp.float32(1e9))
        jmin = jnp.min(t, axis=2, keepdims=True)
        hit = t == jmin
        sels.append(jnp.where(hit, jnp.float32(1.0), jnp.float32(0.0)))
        cur = jnp.where(hit, jnp.float32(-3e38), cur)

    outs = []
    for i in range(VB):
        S = jnp.concatenate([s[i] for s in sels], axis=0)
        g = jnp.dot(S, y1s[i], preferred_element_type=f32)
        mx = jnp.max(g.reshape(K - 1, N, g.shape[1]), axis=0)
        outs.append(_leaky(jnp.maximum(mx, y1s[i]) + y2s[i]))
    return outs


def _dgcnn_kernel(x_ref, a1, b1, a2, b2, a3, b3, a4, b4, w5t, s5, b5, out_ref):
    x0 = [x_ref[i] for i in range(VB)]
    x1 = _edge_layer(x0, a1[...], b1[...])
    x2 = _edge_layer(x1, a2[...], b2[...])
    x3 = _edge_layer(x2, a3[...], b3[...])
    x4 = _edge_layer(x3, a4[...], b4[...])
    for i in range(VB):
        cat = jnp.concatenate([x1[i], x2[i], x3[i], x4[i]], axis=1)
        h5 = jax.lax.dot_general(cat, w5t[...], (((1,), (1,)), ((), ())),
                                 preferred_element_type=f32)
        h5 = _leaky(h5 * s5[...] + b5[...])
        pm = jnp.max(h5, axis=0, keepdims=True)
        pa = jnp.mean(h5, axis=0, keepdims=True)
        out_ref[i] = jnp.concatenate([pm, pa], axis=1)


def _voxhead_kernel(p_ref, l1t, s6, b6, l2t, s7, bt7, valid, out_ref):
    h = jax.lax.dot_general(p_ref[...], l1t[...], (((1,), (1,)), ((), ())),
                            preferred_element_type=f32)
    h = _leaky(h * s6[...] + b6[...])
    h = jax.lax.dot_general(h, l2t[...], (((1,), (1,)), ((), ())),
                            preferred_element_type=f32)
    h = _leaky(h * s7[...] + bt7[...])
    out_ref[...] = h * valid[...]


def _nt(a, b):
    return jax.lax.dot_general(a, b, (((1,), (1,)), ((), ())),
                               preferred_element_type=f32)


def _transformer_kernel(x_in, cnt, ln1g, ln1b, qkvt, aot, aob, ln2g, ln2b,
                        f1t, f1b, f2t, f2b, out_ref):
    l = pl.program_id(0)

    @pl.when(l == 0)
    def _():
        out_ref[...] = x_in[...]

    x = out_ref[...]
    scale = DIM_HEAD ** -0.5

    def ln(v, g, b):
        mu = jnp.mean(v, axis=1, keepdims=True)
        var = jnp.mean((v - mu) ** 2, axis=1, keepdims=True)
        return (v - mu) * jax.lax.rsqrt(var + EPS) * g + b

    h = ln(x, ln1g[0], ln1b[0])
    qkv = _nt(h, qkvt[0])

    counts = cnt[...]
    live = counts > 0
    bparts = []
    for b in range(4):
        r0 = b * NTOK
        hparts = []
        for hh in range(HEADS):
            c = hh * DIM_HEAD
            q = qkv[r0:r0 + NTOK, c:c + DIM_HEAD]
            k_ = qkv[r0:r0 + NTOK, INNER + c:INNER + c + DIM_HEAD]
            v = qkv[r0:r0 + NTOK, 2 * INNER + c:2 * INNER + c + DIM_HEAD]
            dots = jax.lax.dot_general(q, k_, (((1,), (1,)), ((), ())),
                                       preferred_element_type=f32) * scale
            dots = jnp.where(live, dots, jnp.float32(-1e30))
            dots = dots - jnp.max(dots, axis=1, keepdims=True)
            e = jnp.exp(dots) * counts
            attn = e / jnp.sum(e, axis=1, keepdims=True)
            hparts.append(jnp.dot(attn, v, preferred_element_type=f32))
        bparts.append(jnp.concatenate(hparts, axis=1))
    o = jnp.concatenate(bparts, axis=0)
    x = x + _nt(o, aot[0]) + aob[0]

    h = ln(x, ln2g[0], ln2b[0])
    f = _nt(h, f1t[0]) + f1b[0]
    f = 0.5 * f * (1.0 + jax.lax.erf(f * jnp.float32(0.7071067811865476)))
    x = x + _nt(f, f2t[0]) + f2b[0]
    out_ref[...] = x


def _head_kernel(x_ref, w1t, cb1, w2t, cb2, out_ref):
    h = _leaky(jax.lax.dot_general(
        x_ref[...], w1t[...], (((1,), (1,)), ((), ())),
        preferred_element_type=f32) + cb1[...])
    out_ref[...] = jax.lax.dot_general(
        h, w2t[...], (((1,), (1,)), ((), ())),
        preferred_element_type=f32) + cb2[...]


def _const_spec(shape):
    nd = len(shape)
    return pl.BlockSpec(shape, lambda *_: (0,) * nd)


@jax.jit
def kernel(input, cloud_len_list, voxel_sequence, params):
    p = params
    B, MV = input.shape[0], input.shape[1]
    V = B * MV
    x0 = input.reshape(V, N, 3)

    inv = np.float32(1.0 / np.sqrt(1.0 + EPS))

    def split_w(w):
        c = w.shape[1] // 2
        return w[:, :c], w[:, c:] - w[:, :c]

    a1, b1 = split_w(p['conv1_w'])
    a2, b2 = split_w(p['conv2_w'])
    a3, b3 = split_w(p['conv3_w'])
    a4, b4 = split_w(p['conv4_w'])
    w5t = p['conv5_w']
    s5 = (p['bn5_g'] * inv)[None]
    b5 = p['bn5_b'][None]

    dg_in = [x0, a1, b1, a2, b2, a3, b3, a4, b4, w5t, s5, b5]
    dg_specs = [pl.BlockSpec((VB, N, 3), lambda v: (v, 0, 0))]
    dg_specs += [_const_spec(t.shape) for t in dg_in[1:]]
    pooled = pl.pallas_call(
        _dgcnn_kernel,
        grid=(V // VB,),
        compiler_params=pltpu.CompilerParams(
            dimension_semantics=("parallel",)),
        in_specs=dg_specs,
        out_specs=pl.BlockSpec((VB, 1, 1024), lambda v: (v, 0, 0)),
        out_shape=jax.ShapeDtypeStruct((V, 1, 1024), f32),
    )(*dg_in)
    pooled = pooled.reshape(V, 1024)

    lens = cloud_len_list.astype(jnp.int32)
    valid = (jnp.arange(MV, dtype=jnp.int32)[None, :] < lens[:, None])
    valid = valid.reshape(V, 1).astype(f32)
    vh_in = [pooled, p['lin1_w'], (p['bn6_g'] * inv)[None], p['bn6_b'][None],
             p['lin2_w'], (p['bn7_g'] * inv)[None],
             (p['lin2_b'] * inv * p['bn7_g'] + p['bn7_b'])[None], valid]
    vf = pl.pallas_call(
        _voxhead_kernel,
        in_specs=[_const_spec(t.shape) for t in vh_in],
        out_specs=_const_spec((V, DIM)),
        out_shape=jax.ShapeDtypeStruct((V, DIM), f32),
    )(*vh_in)

    cls = jnp.broadcast_to(p['cls_token'], (B, 1, DIM))
    pad = jnp.zeros((B, NTOK - 1 - MV, DIM), f32)
    toks = jnp.concatenate([cls, vf.reshape(B, MV, DIM), pad], axis=1)
    xt = toks.reshape(B * NTOK, DIM)

    def d3(t):
        return t[:, None, :] if t.ndim == 2 else t

    counts = jnp.asarray(
        [[1.0] * (1 + MV) + [float(MAX_VOX - MV)] + [0.0] * (NTOK - MV - 2)],
        dtype=f32)
    tr_in = [xt, counts,
             d3(p['ln1_g']), d3(p['ln1_b']),
             p['qkv_w'],
             p['attn_out_w'], d3(p['attn_out_b']),
             d3(p['ln2_g']), d3(p['ln2_b']),
             p['ff_w1'], d3(p['ff_b1']),
             p['ff_w2'], d3(p['ff_b2'])]
    tr_specs = [_const_spec((B * NTOK, DIM)), _const_spec((1, NTOK))]
    for t in tr_in[2:]:
        tr_specs.append(
            pl.BlockSpec((1,) + t.shape[1:], lambda l: (l, 0, 0)))
    xo = pl.pallas_call(
        _transformer_kernel,
        grid=(DEPTH,),
        in_specs=tr_specs,
        out_specs=_const_spec((B * NTOK, DIM)),
        out_shape=jax.ShapeDtypeStruct((B * NTOK, DIM), f32),
    )(*tr_in)

    cls_f = xo.reshape(B, NTOK, DIM)[:, 0]
    cls_p = jnp.zeros((8, DIM), f32).at[:B].set(cls_f)
    hd_in = [cls_p, p['cls_w1'], p['cls_b1'][None],
             p['cls_w2'], p['cls_b2'][None]]
    logits = pl.pallas_call(
        _head_kernel,
        in_specs=[_const_spec(t.shape) for t in hd_in],
        out_specs=_const_spec((8, NUM_CLASSES)),
        out_shape=jax.ShapeDtypeStruct((8, NUM_CLASSES), f32),
    )(*hd_in)
    return logits[:B]
